# Initial kernel scaffold; baseline (speedup 1.0000x reference)
import jax
import jax.numpy as jnp
from jax.experimental import pallas as pl


def kernel(x, edge_index, W1_l, b1_l, W1_r, W2_l, b2_l, W2_r):
    raise NotImplementedError("write your pallas kernel here")


# speedup vs baseline: 3.5251x; 3.5251x over previous
import functools

import jax
import jax.numpy as jnp
from jax import lax
from jax.experimental import pallas as pl
from jax.experimental.pallas import tpu as pltpu
from jax.experimental.pallas import tpu_sc as plsc

N_NODES = 10000
D = 128
NP = 10240
NC, NS = 2, 16
NW = NC * NS
CHUNK = 128
NCH = 80
ROWS_PER_TILE = NP // NS
DEG_W = 128
BN = 256


def _sc_agg_body(table, src_hbm, dst_hbm, zrows,
                 out_acc, src_v, dst_v, rows_v, acc_sh, sem):
    c = lax.axis_index("c")
    s = lax.axis_index("s")
    wid = c * NS + s
    base = s * ROWS_PER_TILE
    pltpu.sync_copy(zrows, acc_sh.at[pl.ds(base, ROWS_PER_TILE)])
    pltpu.sync_copy(src_hbm.at[wid], src_v)
    pltpu.sync_copy(dst_hbm.at[wid], dst_v)
    plsc.subcore_barrier()

    def step(j, carry):
        pltpu.async_copy(table.at[src_v.at[j]], rows_v, sem).wait()
        pltpu.sync_copy(rows_v, acc_sh.at[dst_v.at[j]], add=True)
        return carry

    lax.fori_loop(0, NCH, step, 0)
    plsc.subcore_barrier()
    pltpu.sync_copy(acc_sh.at[pl.ds(base, ROWS_PER_TILE)],
                    out_acc.at[c, pl.ds(base, ROWS_PER_TILE)])


def _sc_deg_body(dst_hbm, zdeg_hbm, ones_hbm,
                 out_deg, dst_v, ones_v, deg_sh, sem):
    c = lax.axis_index("c")
    s = lax.axis_index("s")
    wid = c * NS + s
    base = s * ROWS_PER_TILE
    pltpu.sync_copy(zdeg_hbm, deg_sh.at[pl.ds(base, ROWS_PER_TILE)])
    pltpu.sync_copy(ones_hbm, ones_v)
    pltpu.sync_copy(dst_hbm.at[wid], dst_v)
    plsc.subcore_barrier()

    def step(j, carry):
        pltpu.sync_copy(ones_v, deg_sh.at[dst_v.at[j]], add=True)
        return carry

    lax.fori_loop(0, NCH, step, 0)
    plsc.subcore_barrier()
    pltpu.sync_copy(deg_sh.at[pl.ds(base, ROWS_PER_TILE)],
                    out_deg.at[c, pl.ds(base, ROWS_PER_TILE)])


def _make_sc_agg():
    mesh = plsc.VectorSubcoreMesh(core_axis_name="c", subcore_axis_name="s", num_cores=NC, num_subcores=NS)
    return pl.kernel(
        _sc_agg_body,
        out_type=jax.ShapeDtypeStruct((NC, NP, D), jnp.float32),
        mesh=mesh,
        scratch_types=[
            pltpu.VMEM((NCH, CHUNK), jnp.int32),
            pltpu.VMEM((NCH, CHUNK), jnp.int32),
            pltpu.VMEM((CHUNK, D), jnp.float32),
            pltpu.VMEM_SHARED((NP, D), jnp.float32),
            pltpu.SemaphoreType.DMA,
        ],
    )


def _make_sc_deg():
    mesh = plsc.VectorSubcoreMesh(core_axis_name="c", subcore_axis_name="s", num_cores=NC, num_subcores=NS)
    return pl.kernel(
        _sc_deg_body,
        out_type=jax.ShapeDtypeStruct((NC, NP, DEG_W), jnp.float32),
        mesh=mesh,
        scratch_types=[
            pltpu.VMEM((NCH, CHUNK), jnp.int32),
            pltpu.VMEM((CHUNK, DEG_W), jnp.float32),
            pltpu.VMEM_SHARED((NP, DEG_W), jnp.float32),
            pltpu.SemaphoreType.DMA,
        ],
    )


def _dense_body(relu, s_ref, deg_ref, x_ref, wl_ref, wr_ref, b_ref, o_ref):
    sacc = s_ref[0] + s_ref[1]
    deg = deg_ref[0, :, :1] + deg_ref[1, :, :1]
    agg = sacc / jnp.maximum(deg, 1.0)
    r = lax.dot_general(agg, wl_ref[...], (((1,), (1,)), ((), ())),
                        preferred_element_type=jnp.float32,
                        precision=lax.Precision.HIGHEST)
    r = r + lax.dot_general(x_ref[...], wr_ref[...], (((1,), (1,)), ((), ())),
                            preferred_element_type=jnp.float32,
                            precision=lax.Precision.HIGHEST)
    r = r + b_ref[...]
    if relu:
        r = jnp.maximum(r, 0.0)
    o_ref[...] = r


def _make_dense(relu):
    return pl.pallas_call(
        functools.partial(_dense_body, relu),
        grid=(NP // BN,),
        in_specs=[
            pl.BlockSpec((NC, BN, D), lambda i: (0, i, 0)),
            pl.BlockSpec((NC, BN, DEG_W), lambda i: (0, i, 0)),
            pl.BlockSpec((BN, D), lambda i: (i, 0)),
            pl.BlockSpec((D, D), lambda i: (0, 0)),
            pl.BlockSpec((D, D), lambda i: (0, 0)),
            pl.BlockSpec((1, D), lambda i: (0, 0)),
        ],
        out_specs=pl.BlockSpec((BN, D), lambda i: (i, 0)),
        out_shape=jax.ShapeDtypeStruct((NP, D), jnp.float32),
    )


_agg = _make_sc_agg()
_deg = _make_sc_deg()
_dense_relu = _make_dense(True)
_dense = _make_dense(False)


def kernel(x, edge_index, W1_l, b1_l, W1_r, W2_l, b2_l, W2_r):
    x_p = jnp.zeros((NP, D), jnp.float32).at[:N_NODES].set(x)
    src = edge_index[0].astype(jnp.int32)
    dst = edge_index[1].astype(jnp.int32)
    n_edges = src.shape[0]
    ep = NW * NCH * CHUNK
    pad = ep - n_edges
    src_p = jnp.concatenate([src, jnp.zeros((pad,), jnp.int32)]).reshape(NW, NCH, CHUNK)
    dst_p = jnp.concatenate([dst, jnp.full((pad,), NP - 1, jnp.int32)]).reshape(NW, NCH, CHUNK)
    zrows = jnp.zeros((ROWS_PER_TILE, D), jnp.float32)
    zdeg = jnp.zeros((ROWS_PER_TILE, DEG_W), jnp.float32)
    ones = jnp.ones((CHUNK, DEG_W), jnp.float32)

    deg = _deg(dst_p, zdeg, ones)
    s1 = _agg(x_p, src_p, dst_p, zrows)
    h = _dense_relu(s1, deg, x_p, W1_l, W1_r, b1_l.reshape(1, D))
    s2 = _agg(h, src_p, dst_p, zrows)
    out = _dense(s2, deg, h, W2_l, W2_r, b2_l.reshape(1, D))
    return out[:N_NODES]


# speedup vs baseline: 7.1436x; 2.0265x over previous
import functools

import jax
import jax.numpy as jnp
from jax import lax
from jax.experimental import pallas as pl
from jax.experimental.pallas import tpu as pltpu
from jax.experimental.pallas import tpu_sc as plsc

N_NODES = 10000
D = 128
NP = 10240
NC, NS = 2, 16
NW = NC * NS
CHUNK = 128
NCH = 80
ROWS_PER_TILE = NP // NS
DEG_W = 128
BN = 256


def _sc_agg_body(table, src_hbm, dst_hbm, zrows,
                 out_acc, src_v, dst_v, rows_v, acc_sh, sem):
    c = lax.axis_index("c")
    s = lax.axis_index("s")
    wid = c * NS + s
    base = s * ROWS_PER_TILE
    pltpu.sync_copy(zrows, acc_sh.at[pl.ds(base, ROWS_PER_TILE)])
    pltpu.sync_copy(src_hbm.at[wid], src_v)
    pltpu.sync_copy(dst_hbm.at[wid], dst_v)
    plsc.subcore_barrier()

    def step(j, carry):
        pltpu.async_copy(table.at[src_v.at[j]], rows_v, sem).wait()
        pltpu.sync_copy(rows_v, acc_sh.at[dst_v.at[j]], add=True)
        return carry

    lax.fori_loop(0, NCH, step, 0)
    plsc.subcore_barrier()
    pltpu.sync_copy(acc_sh.at[pl.ds(base, ROWS_PER_TILE)],
                    out_acc.at[c, pl.ds(base, ROWS_PER_TILE)])


def _sc_deg_body(dst_hbm, zdeg_hbm, ones_hbm,
                 out_deg, dst_v, ones_v, deg_sh, sem):
    c = lax.axis_index("c")
    s = lax.axis_index("s")
    wid = c * NS + s
    base = s * ROWS_PER_TILE
    pltpu.sync_copy(zdeg_hbm, deg_sh.at[pl.ds(base, ROWS_PER_TILE)])
    pltpu.sync_copy(ones_hbm, ones_v)
    pltpu.sync_copy(dst_hbm.at[wid], dst_v)
    plsc.subcore_barrier()

    def step(j, carry):
        pltpu.sync_copy(ones_v, deg_sh.at[dst_v.at[j]], add=True)
        return carry

    lax.fori_loop(0, NCH, step, 0)
    plsc.subcore_barrier()
    pltpu.sync_copy(deg_sh.at[pl.ds(base, ROWS_PER_TILE)],
                    out_deg.at[c, pl.ds(base, ROWS_PER_TILE)])


def _make_sc_agg():
    mesh = plsc.VectorSubcoreMesh(core_axis_name="c", subcore_axis_name="s", num_cores=NC, num_subcores=NS)
    return pl.kernel(
        _sc_agg_body,
        out_type=jax.ShapeDtypeStruct((NC, NP, D), jnp.float32),
        mesh=mesh,
        scratch_types=[
            pltpu.VMEM((NCH, CHUNK), jnp.int32),
            pltpu.VMEM((NCH, CHUNK), jnp.int32),
            pltpu.VMEM((CHUNK, D), jnp.float32),
            pltpu.VMEM_SHARED((NP, D), jnp.float32),
            pltpu.SemaphoreType.DMA,
        ],
    )


def _make_sc_deg():
    mesh = plsc.VectorSubcoreMesh(core_axis_name="c", subcore_axis_name="s", num_cores=NC, num_subcores=NS)
    return pl.kernel(
        _sc_deg_body,
        out_type=jax.ShapeDtypeStruct((NC, NP, DEG_W), jnp.float32),
        mesh=mesh,
        scratch_types=[
            pltpu.VMEM((NCH, CHUNK), jnp.int32),
            pltpu.VMEM((CHUNK, DEG_W), jnp.float32),
            pltpu.VMEM_SHARED((NP, DEG_W), jnp.float32),
            pltpu.SemaphoreType.DMA,
        ],
    )


def _dense_body(relu, s_ref, deg_ref, x_ref, wl_ref, wr_ref, b_ref, o_ref):
    sacc = s_ref[0] + s_ref[1]
    deg = deg_ref[0, :, :1] + deg_ref[1, :, :1]
    agg = sacc / jnp.maximum(deg, 1.0)
    r = lax.dot_general(agg, wl_ref[...], (((1,), (1,)), ((), ())),
                        preferred_element_type=jnp.float32,
                        precision=lax.Precision.HIGHEST)
    r = r + lax.dot_general(x_ref[...], wr_ref[...], (((1,), (1,)), ((), ())),
                            preferred_element_type=jnp.float32,
                            precision=lax.Precision.HIGHEST)
    r = r + b_ref[...]
    if relu:
        r = jnp.maximum(r, 0.0)
    o_ref[...] = r


def _make_dense(relu):
    return pl.pallas_call(
        functools.partial(_dense_body, relu),
        grid=(NP // BN,),
        in_specs=[
            pl.BlockSpec((NC, BN, D), lambda i: (0, i, 0)),
            pl.BlockSpec((NC, BN, DEG_W), lambda i: (0, i, 0)),
            pl.BlockSpec((BN, D), lambda i: (i, 0)),
            pl.BlockSpec((D, D), lambda i: (0, 0)),
            pl.BlockSpec((D, D), lambda i: (0, 0)),
            pl.BlockSpec((1, D), lambda i: (0, 0)),
        ],
        out_specs=pl.BlockSpec((BN, D), lambda i: (i, 0)),
        out_shape=jax.ShapeDtypeStruct((NP, D), jnp.float32),
    )


_agg = _make_sc_agg()
_deg = _make_sc_deg()
_dense_relu = _make_dense(True)
_dense = _make_dense(False)


def kernel(x, edge_index, W1_l, b1_l, W1_r, W2_l, b2_l, W2_r):
    x_p = jnp.zeros((NP, D), jnp.float32).at[:N_NODES].set(x)
    src = edge_index[0].astype(jnp.int32)
    dst = edge_index[1].astype(jnp.int32)
    n_edges = src.shape[0]
    ep = NW * NCH * CHUNK
    pad = ep - n_edges
    pad_src = jnp.arange(pad, dtype=jnp.int32) % N_NODES
    pad_dst = N_NODES + (jnp.arange(pad, dtype=jnp.int32) % (NP - N_NODES))
    src_p = jnp.concatenate([src, pad_src]).reshape(NW, NCH, CHUNK)
    dst_p = jnp.concatenate([dst, pad_dst]).reshape(NW, NCH, CHUNK)
    zrows = jnp.zeros((ROWS_PER_TILE, D), jnp.float32)
    zdeg = jnp.zeros((ROWS_PER_TILE, DEG_W), jnp.float32)
    ones = jnp.ones((CHUNK, DEG_W), jnp.float32)

    deg = _deg(dst_p, zdeg, ones)
    s1 = _agg(x_p, src_p, dst_p, zrows)
    h = _dense_relu(s1, deg, x_p, W1_l, W1_r, b1_l.reshape(1, D))
    s2 = _agg(h, src_p, dst_p, zrows)
    out = _dense(s2, deg, h, W2_l, W2_r, b2_l.reshape(1, D))
    return out[:N_NODES]
